# Initial kernel scaffold; baseline (speedup 1.0000x reference)
import functools

import jax
import jax.numpy as jnp
from jax import lax
from jax.experimental import pallas as pl
from jax.experimental.pallas import tpu as pltpu
from jax.experimental.pallas import tpu_sc as plsc

_NC = 2
_NS = 16
_NW = _NC * _NS
_L = 16
_D = 128
_E = 320000
_BPW = _E // _NW
_CHUNK = 80
_NCHUNK = _BPW // _CHUNK


def _dot_chunk(rows_u, rows_m, tscr, out_v, c):
    lane = lax.iota(jnp.int32, (_L,))
    for g in range(_CHUNK // _L):
        for j in range(_L):
            e = g * _L + j
            acc = rows_u[e, pl.ds(0, _L)] * rows_m[e, pl.ds(0, _L)]
            for d in range(1, _D // _L):
                acc += rows_u[e, pl.ds(d * _L, _L)] * rows_m[e, pl.ds(d * _L, _L)]
            plsc.store_scatter(tscr, [lane, jnp.full((_L,), j, jnp.int32)], acc)
        tot = tscr[0, :]
        for r in range(1, _L):
            tot = tot + tscr[r, :]
        out_v[pl.ds(c * _CHUNK + g * _L, _L)] = tot


def _sc_body(xu, xm, eidx, out, idx_u, idx_m, rows_u, rows_m, tscr, out_v,
             sem_u, sem_m):
    wid = lax.axis_index("s") * _NC + lax.axis_index("c")
    base = wid * _BPW
    pltpu.sync_copy(eidx.at[0, pl.ds(base, _BPW)], idx_u)
    pltpu.sync_copy(eidx.at[1, pl.ds(base, _BPW)], idx_m)

    def step(c, carry):
        cu = pltpu.async_copy(xu.at[idx_u.at[pl.ds(c * _CHUNK, _CHUNK)]],
                              rows_u, sem_u)
        cm = pltpu.async_copy(xm.at[idx_m.at[pl.ds(c * _CHUNK, _CHUNK)]],
                              rows_m, sem_m)
        cu.wait()
        cm.wait()
        _dot_chunk(rows_u, rows_m, tscr, out_v, c)
        return carry

    lax.fori_loop(0, _NCHUNK, step, 0)
    pltpu.sync_copy(out_v, out.at[pl.ds(base, _BPW)])


_sc_kernel = functools.partial(
    pl.kernel,
    out_type=jax.ShapeDtypeStruct((_E,), jnp.float32),
    mesh=plsc.VectorSubcoreMesh(core_axis_name="c", subcore_axis_name="s",
                                num_cores=_NC, num_subcores=_NS),
    scratch_types=[
        pltpu.VMEM((_BPW,), jnp.int32),
        pltpu.VMEM((_BPW,), jnp.int32),
        pltpu.VMEM((_CHUNK, _D), jnp.float32),
        pltpu.VMEM((_CHUNK, _D), jnp.float32),
        pltpu.VMEM((_L, _L), jnp.float32),
        pltpu.VMEM((_BPW,), jnp.float32),
        pltpu.SemaphoreType.DMA,
        pltpu.SemaphoreType.DMA,
    ],
)(_sc_body)


def kernel(x_user, x_movie, edge_label_index):
    return _sc_kernel(x_user, x_movie, edge_label_index)


# speedup vs baseline: 2.5056x; 2.5056x over previous
import functools

import jax
import jax.numpy as jnp
from jax import lax
from jax.experimental import pallas as pl
from jax.experimental.pallas import tpu as pltpu
from jax.experimental.pallas import tpu_sc as plsc

_NC = 2
_NS = 16
_NW = _NC * _NS
_L = 16
_D = 128
_E = 320000
_BPW = _E // _NW
_CHUNK = 80
_NCHUNK = _BPW // _CHUNK


def _lane_perm(x, idx):
    dnums = lax.GatherDimensionNumbers(
        offset_dims=(), collapsed_slice_dims=(0,), start_index_map=(0,))
    return lax.gather(x, idx[:, None], dnums, slice_sizes=(1,),
                      mode=lax.GatherScatterMode.PROMISE_IN_BOUNDS)


def _dot_chunk(rows_u, rows_m, out_v, c):
    lane = lax.iota(jnp.int32, _L)
    for g in range(_CHUNK // _L):
        tot = jnp.zeros((_L,), jnp.float32)
        for j in range(_L):
            e = g * _L + j
            acc = rows_u[e, pl.ds(0, _L)] * rows_m[e, pl.ds(0, _L)]
            for d in range(1, _D // _L):
                acc += rows_u[e, pl.ds(d * _L, _L)] * rows_m[e, pl.ds(d * _L, _L)]
            for k in (8, 4, 2, 1):
                acc = acc + _lane_perm(acc, lane ^ k)
            tot = jnp.where(lane == j, acc, tot)
        out_v[pl.ds(c * _CHUNK + g * _L, _L)] = tot


def _sc_body(xu, xm, eidx, out, idx_u, idx_m, rows_u, rows_m, out_v,
             sem_u, sem_m):
    wid = lax.axis_index("s") * _NC + lax.axis_index("c")
    base = wid * _BPW
    pltpu.sync_copy(eidx.at[pl.ds(base, _BPW)], idx_u)
    pltpu.sync_copy(eidx.at[pl.ds(_E + base, _BPW)], idx_m)

    def step(c, carry):
        cu = pltpu.async_copy(xu.at[idx_u.at[pl.ds(c * _CHUNK, _CHUNK)]],
                              rows_u, sem_u)
        cm = pltpu.async_copy(xm.at[idx_m.at[pl.ds(c * _CHUNK, _CHUNK)]],
                              rows_m, sem_m)
        cu.wait()
        cm.wait()
        _dot_chunk(rows_u, rows_m, out_v, c)
        return carry

    lax.fori_loop(0, _NCHUNK, step, 0)
    pltpu.sync_copy(out_v, out.at[pl.ds(base, _BPW)])


_sc_kernel = functools.partial(
    pl.kernel,
    out_type=jax.ShapeDtypeStruct((_E,), jnp.float32),
    mesh=plsc.VectorSubcoreMesh(core_axis_name="c", subcore_axis_name="s",
                                num_cores=_NC, num_subcores=_NS),
    scratch_types=[
        pltpu.VMEM((_BPW,), jnp.int32),
        pltpu.VMEM((_BPW,), jnp.int32),
        pltpu.VMEM((_CHUNK, _D), jnp.float32),
        pltpu.VMEM((_CHUNK, _D), jnp.float32),
        pltpu.VMEM((_BPW,), jnp.float32),
        pltpu.SemaphoreType.DMA,
        pltpu.SemaphoreType.DMA,
    ],
)(_sc_body)


def kernel(x_user, x_movie, edge_label_index):
    return _sc_kernel(x_user, x_movie, edge_label_index.reshape(-1))
